# baseline (speedup 1.0000x reference)
import functools

import jax
import jax.numpy as jnp
from jax.experimental import pallas as pl
from jax.experimental.pallas import tpu as pltpu

C = 16
EPS = 0.001
_STRIDES = [1, 1, 2, 1, 1, 2, 1, 1, 1, 2, 1, 1, 1, 2, 1, 1, 1]

_PREC = jax.lax.Precision.HIGHEST


def _mm(a, b):
    return jax.lax.dot_general(
        a, b, (((1,), (0,)), ((), ())),
        preferred_element_type=jnp.float32, precision=_PREC)


def _kw_combine(acc, wcol, W):
    zm1 = acc[:, 0:C]
    z0 = acc[:, C:2 * C]
    zp1 = acc[:, 2 * C:3 * C]
    zrow = jnp.zeros((1, C), dtype=acc.dtype)
    sm = jnp.concatenate([zrow, zm1[:-1]], axis=0)
    sp = jnp.concatenate([zp1[1:], zrow], axis=0)
    return z0 + jnp.where(wcol == 0, 0.0, sm) + jnp.where(wcol == W - 1, 0.0, sp)


def _plane_out(p0, p1, p2, wmat, bias, H, W):
    HW = H * W
    blocks = []
    for p in (p0, p1, p2):
        z = jnp.zeros((W, C), dtype=p.dtype)
        pm = jnp.concatenate([z, p[: HW - W]], axis=0)
        pp = jnp.concatenate([p[W:], z], axis=0)
        blocks += [pm, p, pp]
    x9 = jnp.concatenate(blocks, axis=1)
    acc = _mm(x9, wmat)
    r = jax.lax.broadcasted_iota(jnp.int32, (HW, 1), 0)
    y = _kw_combine(acc, r % W, W)
    return jnp.maximum(y + bias, 0.0)


def _big_body(*refs, H, W, stride, pps):
    n_in = pps + 2 if stride == 1 else 2 * pps + 1
    in_refs = refs[:n_in]
    w_ref, b_ref, y_ref = refs[n_in], refs[n_in + 1], refs[n_in + 2]
    wmat = w_ref[...]
    bias = b_ref[0]
    for j in range(pps):
        k = j if stride == 1 else 2 * j
        y = _plane_out(in_refs[k][0], in_refs[k + 1][0], in_refs[k + 2][0],
                       wmat, bias, H, W)
        y_ref[j] = y


def _conv_big(xpad, w9, b, D_out, H, W, stride, pps):
    HW = H * W
    n_in = pps + 2 if stride == 1 else 2 * pps + 1
    step = pps * stride
    body = functools.partial(_big_body, H=H, W=W, stride=stride, pps=pps)
    in_specs = [
        pl.BlockSpec((1, HW, C), lambda d, kk=k: (step * d + kk, 0, 0))
        for k in range(n_in)
    ]
    in_specs.append(pl.BlockSpec((9 * C, 3 * C), lambda d: (0, 0)))
    in_specs.append(pl.BlockSpec((1, C), lambda d: (0, 0)))
    return pl.pallas_call(
        body,
        grid=(D_out // pps,),
        in_specs=in_specs,
        out_specs=pl.BlockSpec((pps, HW, C), lambda d: (d, 0, 0)),
        out_shape=jax.ShapeDtypeStruct((D_out, HW, C), jnp.float32),
    )(*([xpad] * n_in), w9, b)


def _layer_global(xa, wmat, bias, D, H, W):
    HW = H * W
    M = D * HW
    r = jax.lax.broadcasted_iota(jnp.int32, (M, 1), 0)
    hrow = (r // W) % H
    wcol = r % W
    blocks = []
    for kd in range(3):
        for kh in range(3):
            start = W + kd * HW + (kh - 1) * W
            blk = jax.lax.slice_in_dim(xa, start, start + M, axis=0)
            if kh == 0:
                blk = jnp.where(hrow == 0, 0.0, blk)
            elif kh == 2:
                blk = jnp.where(hrow == H - 1, 0.0, blk)
            blocks.append(blk)
    x9 = jnp.concatenate(blocks, axis=1)
    acc = _mm(x9, wmat)
    y = _kw_combine(acc, wcol, W)
    return jnp.maximum(y + bias, 0.0)


def _subsample(y, D, H, W):
    t = y.reshape(D * H * (W // 2), 2, C)[:, 0, :]
    t = t.reshape(D * (H // 2), 2 * (W // 2), C)[:, 0:W // 2, :]
    t = t.reshape(D // 2, 2 * (H // 2) * (W // 2), C)
    t = t[:, 0:(H // 2) * (W // 2), :]
    return t.reshape((D // 2) * (H // 2) * (W // 2), C)


def _fused_body(x_ref, w_ref, b_ref, o2_ref, o3_ref, o4_ref, arena):
    D = H = W = 16
    h = x_ref[...]
    for i, s in enumerate(_STRIDES[6:]):
        HW = H * W
        M = D * HW
        AR = W + (D + 2) * HW + W
        arena[pl.ds(0, AR)] = jnp.zeros((AR, C), jnp.float32)
        arena[pl.ds(W + HW, M)] = h
        xa = arena[pl.ds(0, AR)]
        y = _layer_global(xa, w_ref[i], b_ref[i, 0], D, H, W)
        if s == 2:
            y = _subsample(y, D, H, W)
            D //= 2
            H //= 2
            W //= 2
        h = y
        layer_idx = 6 + i
        if layer_idx == 8:
            o2_ref[...] = h
        elif layer_idx == 12:
            o3_ref[...] = h
        elif layer_idx == 16:
            o4_ref[...] = h


def _fused_tail(x16, w_all, b_all):
    ar0 = 16 + 18 * 256 + 16
    return pl.pallas_call(
        _fused_body,
        in_specs=[
            pl.BlockSpec((16 ** 3, C), lambda: (0, 0)),
            pl.BlockSpec((11, 9 * C, 3 * C), lambda: (0, 0, 0)),
            pl.BlockSpec((11, 1, C), lambda: (0, 0, 0)),
        ],
        out_specs=[
            pl.BlockSpec((16 ** 3, C), lambda: (0, 0)),
            pl.BlockSpec((8 ** 3, C), lambda: (0, 0)),
            pl.BlockSpec((4 ** 3, C), lambda: (0, 0)),
        ],
        out_shape=[
            jax.ShapeDtypeStruct((16 ** 3, C), jnp.float32),
            jax.ShapeDtypeStruct((8 ** 3, C), jnp.float32),
            jax.ShapeDtypeStruct((4 ** 3, C), jnp.float32),
        ],
        scratch_shapes=[pltpu.VMEM((ar0, C), jnp.float32)],
    )(x16, w_all, b_all)


def _fold_w(w, g):
    inv = 1.0 / jnp.sqrt(1.0 + EPS)
    wS = w * (inv * g)[:, None, None, None, None]
    return jnp.transpose(wS, (2, 3, 1, 4, 0)).reshape(9 * C, 3 * C)


def kernel(x, params):
    h = jnp.transpose(x[0], (1, 2, 3, 0)).reshape(64, 64 * 64, C)
    D = H = W = 64
    outs = []
    for i in range(6):
        (w, g, b), s = params[i], _STRIDES[i]
        w9 = _fold_w(w, g)
        xpad = jnp.pad(h, ((1, 1), (0, 0), (0, 0)))
        D_out = D if s == 1 else D // 2
        pps = 4 if s == 1 else 2
        y = _conv_big(xpad, w9, b.reshape(1, C), D_out, H, W, s, pps)
        if s == 2:
            y = y.reshape(D_out, H, W, C)[:, ::2, ::2]
            H //= 2
            W //= 2
            y = y.reshape(D_out, H * W, C)
        D = D_out
        h = y
        if i == 4:
            outs.append(jnp.transpose(h.reshape(D, H, W, C), (3, 0, 1, 2))[None])
    w_all = jnp.stack([_fold_w(params[i][0], params[i][1]) for i in range(6, 17)])
    b_all = jnp.stack([params[i][2].reshape(1, C) for i in range(6, 17)])
    o2, o3, o4 = _fused_tail(h.reshape(16 ** 3, C), w_all, b_all)
    outs.append(jnp.transpose(o2.reshape(16, 16, 16, C), (3, 0, 1, 2))[None])
    outs.append(jnp.transpose(o3.reshape(8, 8, 8, C), (3, 0, 1, 2))[None])
    outs.append(jnp.transpose(o4.reshape(4, 4, 4, C), (3, 0, 1, 2))[None])
    return outs[0], outs[1], outs[2], outs[3]


# speedup vs baseline: 2.1511x; 2.1511x over previous
import functools

import jax
import jax.numpy as jnp
from jax.experimental import pallas as pl
from jax.experimental.pallas import tpu as pltpu

C = 16
EPS = 0.001
_STRIDES = [1, 1, 2, 1, 1, 2, 1, 1, 1, 2, 1, 1, 1, 2, 1, 1, 1]

_PREC = jax.lax.Precision.HIGHEST


def _mm(a, b):
    return jax.lax.dot_general(
        a, b, (((1,), (0,)), ((), ())),
        preferred_element_type=jnp.float32, precision=_PREC)


def _kw_combine(acc, wcol, W):
    zm1 = acc[:, 0:C]
    z0 = acc[:, C:2 * C]
    zp1 = acc[:, 2 * C:3 * C]
    zrow = jnp.zeros((1, C), dtype=acc.dtype)
    sm = jnp.concatenate([zrow, zm1[:-1]], axis=0)
    sp = jnp.concatenate([zp1[1:], zrow], axis=0)
    return z0 + jnp.where(wcol == 0, 0.0, sm) + jnp.where(wcol == W - 1, 0.0, sp)


def _packed_body(p0_ref, p1_ref, p2_ref, w_ref, b_ref, y_ref, *, H, Wb):
    R = H * Wb
    r = jax.lax.broadcasted_iota(jnp.int32, (R, 1), 0)
    wbcol = r % Wb
    zrow = jnp.zeros((1, 8 * C), jnp.float32)
    acc = None
    t = 0
    for p_ref in (p0_ref, p1_ref, p2_ref):
        p = p_ref[0]
        for kh in range(3):
            if kh == 0:
                xs = jnp.concatenate(
                    [jnp.zeros((Wb, 8 * C), jnp.float32), p[: R - Wb]], axis=0)
            elif kh == 1:
                xs = p
            else:
                xs = jnp.concatenate(
                    [p[Wb:], jnp.zeros((Wb, 8 * C), jnp.float32)], axis=0)
            om = jnp.where(wbcol == 0, 0.0,
                           jnp.concatenate([zrow, xs[:-1]], axis=0))
            op = jnp.where(wbcol == Wb - 1, 0.0,
                           jnp.concatenate([xs[1:], zrow], axis=0))
            part = (_mm(xs, w_ref[3 * t]) + _mm(om, w_ref[3 * t + 1])
                    + _mm(op, w_ref[3 * t + 2]))
            acc = part if acc is None else acc + part
            t += 1
    y_ref[0] = jnp.maximum(acc + b_ref[0], 0.0)


def _conv_packed(xpad, wbig, bt, D_out, H, Wb, stride):
    R = H * Wb
    body = functools.partial(_packed_body, H=H, Wb=Wb)
    s = stride
    in_specs = [
        pl.BlockSpec((1, R, 8 * C), lambda d, kk=k: (s * d + kk, 0, 0))
        for k in range(3)
    ]
    in_specs.append(pl.BlockSpec((27, 8 * C, 8 * C), lambda d: (0, 0, 0)))
    in_specs.append(pl.BlockSpec((1, 8 * C), lambda d: (0, 0)))
    return pl.pallas_call(
        body,
        grid=(D_out,),
        in_specs=in_specs,
        out_specs=pl.BlockSpec((1, R, 8 * C), lambda d: (d, 0, 0)),
        out_shape=jax.ShapeDtypeStruct((D_out, R, 8 * C), jnp.float32),
    )(xpad, xpad, xpad, wbig, bt)


_SEL = None


def _band_sel():
    global _SEL
    if _SEL is None:
        import numpy as np
        sel = np.zeros((8, 8, 3), dtype=np.float32)
        for j in range(8):
            for i in range(8):
                kw = j - i + 1
                if 0 <= kw <= 2:
                    sel[j, i, kw] = 1.0
        _SEL = jnp.asarray(sel)
    return _SEL


def _fold_w_packed(w, g):
    inv = 1.0 / jnp.sqrt(1.0 + EPS)
    wS = w * (inv * g)[:, None, None, None, None]
    wT = jnp.transpose(wS, (2, 3, 4, 1, 0))
    wT = wT.reshape(9, 3, C, C)
    sel = _band_sel()
    b0 = jnp.einsum("jik,tkab->tjaib", sel, wT).reshape(9, 8 * C, 8 * C)
    bm = jnp.zeros((9, 8 * C, 8 * C), jnp.float32)
    bm = bm.at[:, 7 * C:8 * C, 0:C].set(wT[:, 0])
    bp = jnp.zeros((9, 8 * C, 8 * C), jnp.float32)
    bp = bp.at[:, 0:C, 7 * C:8 * C].set(wT[:, 2])
    return jnp.stack([b0, bm, bp], axis=1).reshape(27, 8 * C, 8 * C)


def _layer_global(xa, wmat, bias, D, H, W):
    HW = H * W
    M = D * HW
    r = jax.lax.broadcasted_iota(jnp.int32, (M, 1), 0)
    hrow = (r // W) % H
    wcol = r % W
    blocks = []
    for kd in range(3):
        for kh in range(3):
            start = W + kd * HW + (kh - 1) * W
            blk = jax.lax.slice_in_dim(xa, start, start + M, axis=0)
            if kh == 0:
                blk = jnp.where(hrow == 0, 0.0, blk)
            elif kh == 2:
                blk = jnp.where(hrow == H - 1, 0.0, blk)
            blocks.append(blk)
    x9 = jnp.concatenate(blocks, axis=1)
    acc = _mm(x9, wmat)
    y = _kw_combine(acc, wcol, W)
    return jnp.maximum(y + bias, 0.0)


def _subsample(y, D, H, W):
    t = y.reshape(D * H * (W // 2), 2, C)[:, 0, :]
    t = t.reshape(D * (H // 2), 2 * (W // 2), C)[:, 0:W // 2, :]
    t = t.reshape(D // 2, 2 * (H // 2) * (W // 2), C)
    t = t[:, 0:(H // 2) * (W // 2), :]
    return t.reshape((D // 2) * (H // 2) * (W // 2), C)


def _fused_body(x_ref, w_ref, b_ref, o2_ref, o3_ref, o4_ref, arena):
    D = H = W = 16
    h = x_ref[...]
    for i, s in enumerate(_STRIDES[6:]):
        HW = H * W
        M = D * HW
        AR = W + (D + 2) * HW + W
        arena[pl.ds(0, AR)] = jnp.zeros((AR, C), jnp.float32)
        arena[pl.ds(W + HW, M)] = h
        xa = arena[pl.ds(0, AR)]
        y = _layer_global(xa, w_ref[i], b_ref[i, 0], D, H, W)
        if s == 2:
            y = _subsample(y, D, H, W)
            D //= 2
            H //= 2
            W //= 2
        h = y
        layer_idx = 6 + i
        if layer_idx == 8:
            o2_ref[...] = h
        elif layer_idx == 12:
            o3_ref[...] = h
        elif layer_idx == 16:
            o4_ref[...] = h


def _fused_tail(x16, w_all, b_all):
    ar0 = 16 + 18 * 256 + 16
    return pl.pallas_call(
        _fused_body,
        in_specs=[
            pl.BlockSpec((16 ** 3, C), lambda: (0, 0)),
            pl.BlockSpec((11, 9 * C, 3 * C), lambda: (0, 0, 0)),
            pl.BlockSpec((11, 1, C), lambda: (0, 0, 0)),
        ],
        out_specs=[
            pl.BlockSpec((16 ** 3, C), lambda: (0, 0)),
            pl.BlockSpec((8 ** 3, C), lambda: (0, 0)),
            pl.BlockSpec((4 ** 3, C), lambda: (0, 0)),
        ],
        out_shape=[
            jax.ShapeDtypeStruct((16 ** 3, C), jnp.float32),
            jax.ShapeDtypeStruct((8 ** 3, C), jnp.float32),
            jax.ShapeDtypeStruct((4 ** 3, C), jnp.float32),
        ],
        scratch_shapes=[pltpu.VMEM((ar0, C), jnp.float32)],
    )(x16, w_all, b_all)


def _fold_w(w, g):
    inv = 1.0 / jnp.sqrt(1.0 + EPS)
    wS = w * (inv * g)[:, None, None, None, None]
    return jnp.transpose(wS, (2, 3, 1, 4, 0)).reshape(9 * C, 3 * C)


def kernel(x, params):
    D = H = W = 64
    h = jnp.transpose(x[0], (1, 2, 3, 0)).reshape(D, H * (W // 8), 8 * C)
    outs = []
    for i in range(6):
        (w, g, b), s = params[i], _STRIDES[i]
        wbig = _fold_w_packed(w, g)
        bt = jnp.tile(b, 8).reshape(1, 8 * C)
        xpad = jnp.pad(h, ((1, 1), (0, 0), (0, 0)))
        D_out = D if s == 1 else D // 2
        y = _conv_packed(xpad, wbig, bt, D_out, H, W // 8, s)
        if s == 2:
            y = y.reshape(D_out, H, W // 8, 8, C)[:, ::2, :, ::2]
            H //= 2
            W //= 2
            y = y.reshape(D_out, H * (W // 8), 8 * C)
        D = D_out
        h = y
        if i == 4:
            t = h.reshape(D, H, W // 8, 8, C).reshape(D, H, W, C)
            outs.append(jnp.transpose(t, (3, 0, 1, 2))[None])
    w_all = jnp.stack([_fold_w(params[i][0], params[i][1]) for i in range(6, 17)])
    b_all = jnp.stack([params[i][2].reshape(1, C) for i in range(6, 17)])
    x16 = h.reshape(16, 16, 2, 8, C).reshape(16 ** 3, C)
    o2, o3, o4 = _fused_tail(x16, w_all, b_all)
    outs.append(jnp.transpose(o2.reshape(16, 16, 16, C), (3, 0, 1, 2))[None])
    outs.append(jnp.transpose(o3.reshape(8, 8, 8, C), (3, 0, 1, 2))[None])
    outs.append(jnp.transpose(o4.reshape(4, 4, 4, C), (3, 0, 1, 2))[None])
    return outs[0], outs[1], outs[2], outs[3]


# speedup vs baseline: 2.6496x; 1.2317x over previous
import functools

import jax
import jax.numpy as jnp
from jax.experimental import pallas as pl
from jax.experimental.pallas import tpu as pltpu

C = 16
EPS = 0.001
_STRIDES = [1, 1, 2, 1, 1, 2, 1, 1, 1, 2, 1, 1, 1, 2, 1, 1, 1]

_PREC = jax.lax.Precision.HIGHEST


def _mm(a, b):
    return jax.lax.dot_general(
        a, b, (((1,), (0,)), ((), ())),
        preferred_element_type=jnp.float32, precision=_PREC)


def _kw_combine(acc, wcol, W):
    zm1 = acc[:, 0:C]
    z0 = acc[:, C:2 * C]
    zp1 = acc[:, 2 * C:3 * C]
    zrow = jnp.zeros((1, C), dtype=acc.dtype)
    sm = jnp.concatenate([zrow, zm1[:-1]], axis=0)
    sp = jnp.concatenate([zp1[1:], zrow], axis=0)
    return z0 + jnp.where(wcol == 0, 0.0, sm) + jnp.where(wcol == W - 1, 0.0, sp)


def _packed_body(p0_ref, p1_ref, p2_ref, w_ref, b_ref, y_ref, *, H, Wb):
    R = H * Wb
    r = jax.lax.broadcasted_iota(jnp.int32, (R, 1), 0)
    wbcol = r % Wb
    lcol = jax.lax.broadcasted_iota(jnp.int32, (1, 8 * C), 1)
    m_dn = (wbcol != 0) & (lcol >= 7 * C)
    m_up = (wbcol != Wb - 1) & (lcol < C)
    zrow = jnp.zeros((1, 8 * C), jnp.float32)
    acc = None
    t = 0
    for p_ref in (p0_ref, p1_ref, p2_ref):
        p = p_ref[0]
        for kh in range(3):
            if kh == 0:
                xs = jnp.concatenate(
                    [jnp.zeros((Wb, 8 * C), jnp.float32), p[: R - Wb]], axis=0)
            elif kh == 1:
                xs = p
            else:
                xs = jnp.concatenate(
                    [p[Wb:], jnp.zeros((Wb, 8 * C), jnp.float32)], axis=0)
            sd = jnp.concatenate([zrow, xs[:-1]], axis=0)
            su = jnp.concatenate([xs[1:], zrow], axis=0)
            omp = jnp.where(m_dn, sd, 0.0) + jnp.where(m_up, su, 0.0)
            part = _mm(xs, w_ref[2 * t]) + _mm(omp, w_ref[2 * t + 1])
            acc = part if acc is None else acc + part
            t += 1
    y_ref[0] = jnp.maximum(acc + b_ref[0], 0.0)


def _conv_packed(xpad, wbig, bt, D_out, H, Wb, stride):
    R = H * Wb
    body = functools.partial(_packed_body, H=H, Wb=Wb)
    s = stride
    in_specs = [
        pl.BlockSpec((1, R, 8 * C), lambda d, kk=k: (s * d + kk, 0, 0))
        for k in range(3)
    ]
    in_specs.append(pl.BlockSpec((18, 8 * C, 8 * C), lambda d: (0, 0, 0)))
    in_specs.append(pl.BlockSpec((1, 8 * C), lambda d: (0, 0)))
    return pl.pallas_call(
        body,
        grid=(D_out,),
        in_specs=in_specs,
        out_specs=pl.BlockSpec((1, R, 8 * C), lambda d: (d, 0, 0)),
        out_shape=jax.ShapeDtypeStruct((D_out, R, 8 * C), jnp.float32),
    )(xpad, xpad, xpad, wbig, bt)


_SEL = None


def _band_sel():
    global _SEL
    if _SEL is None:
        import numpy as np
        sel = np.zeros((8, 8, 3), dtype=np.float32)
        for j in range(8):
            for i in range(8):
                kw = j - i + 1
                if 0 <= kw <= 2:
                    sel[j, i, kw] = 1.0
        _SEL = jnp.asarray(sel)
    return _SEL


def _fold_w_packed(w, g):
    inv = 1.0 / jnp.sqrt(1.0 + EPS)
    wS = w * (inv * g)[:, None, None, None, None]
    wT = jnp.transpose(wS, (2, 3, 4, 1, 0))
    wT = wT.reshape(9, 3, C, C)
    sel = _band_sel()
    b0 = jnp.einsum("jik,tkab->tjaib", sel, wT).reshape(9, 8 * C, 8 * C)
    bmp = jnp.zeros((9, 8 * C, 8 * C), jnp.float32)
    bmp = bmp.at[:, 7 * C:8 * C, 0:C].set(wT[:, 0])
    bmp = bmp.at[:, 0:C, 7 * C:8 * C].set(wT[:, 2])
    return jnp.stack([b0, bmp], axis=1).reshape(18, 8 * C, 8 * C)


def _layer_global(xa, wmat, bias, D, H, W):
    HW = H * W
    M = D * HW
    r = jax.lax.broadcasted_iota(jnp.int32, (M, 1), 0)
    hrow = (r // W) % H
    wcol = r % W
    blocks = []
    for kd in range(3):
        for kh in range(3):
            start = W + kd * HW + (kh - 1) * W
            blk = jax.lax.slice_in_dim(xa, start, start + M, axis=0)
            if kh == 0:
                blk = jnp.where(hrow == 0, 0.0, blk)
            elif kh == 2:
                blk = jnp.where(hrow == H - 1, 0.0, blk)
            blocks.append(blk)
    x9 = jnp.concatenate(blocks, axis=1)
    acc = _mm(x9, wmat)
    y = _kw_combine(acc, wcol, W)
    return jnp.maximum(y + bias, 0.0)


def _subsample(y, D, H, W):
    t = y.reshape(D * H * (W // 2), 2, C)[:, 0, :]
    t = t.reshape(D * (H // 2), 2 * (W // 2), C)[:, 0:W // 2, :]
    t = t.reshape(D // 2, 2 * (H // 2) * (W // 2), C)
    t = t[:, 0:(H // 2) * (W // 2), :]
    return t.reshape((D // 2) * (H // 2) * (W // 2), C)


def _fused_body(x_ref, w_ref, b_ref, o2_ref, o3_ref, o4_ref, arena):
    D = H = W = 16
    h = x_ref[...]
    for i, s in enumerate(_STRIDES[6:]):
        HW = H * W
        M = D * HW
        AR = W + (D + 2) * HW + W
        arena[pl.ds(0, AR)] = jnp.zeros((AR, C), jnp.float32)
        arena[pl.ds(W + HW, M)] = h
        xa = arena[pl.ds(0, AR)]
        y = _layer_global(xa, w_ref[i], b_ref[i, 0], D, H, W)
        if s == 2:
            y = _subsample(y, D, H, W)
            D //= 2
            H //= 2
            W //= 2
        h = y
        layer_idx = 6 + i
        if layer_idx == 8:
            o2_ref[...] = h
        elif layer_idx == 12:
            o3_ref[...] = h
        elif layer_idx == 16:
            o4_ref[...] = h


def _fused_tail(x16, w_all, b_all):
    ar0 = 16 + 18 * 256 + 16
    return pl.pallas_call(
        _fused_body,
        in_specs=[
            pl.BlockSpec((16 ** 3, C), lambda: (0, 0)),
            pl.BlockSpec((11, 9 * C, 3 * C), lambda: (0, 0, 0)),
            pl.BlockSpec((11, 1, C), lambda: (0, 0, 0)),
        ],
        out_specs=[
            pl.BlockSpec((16 ** 3, C), lambda: (0, 0)),
            pl.BlockSpec((8 ** 3, C), lambda: (0, 0)),
            pl.BlockSpec((4 ** 3, C), lambda: (0, 0)),
        ],
        out_shape=[
            jax.ShapeDtypeStruct((16 ** 3, C), jnp.float32),
            jax.ShapeDtypeStruct((8 ** 3, C), jnp.float32),
            jax.ShapeDtypeStruct((4 ** 3, C), jnp.float32),
        ],
        scratch_shapes=[pltpu.VMEM((ar0, C), jnp.float32)],
    )(x16, w_all, b_all)


def _fold_w(w, g):
    inv = 1.0 / jnp.sqrt(1.0 + EPS)
    wS = w * (inv * g)[:, None, None, None, None]
    return jnp.transpose(wS, (2, 3, 1, 4, 0)).reshape(9 * C, 3 * C)


def kernel(x, params):
    D = H = W = 64
    h = jnp.transpose(x[0], (1, 2, 3, 0)).reshape(D, H * (W // 8), 8 * C)
    outs = []
    for i in range(6):
        (w, g, b), s = params[i], _STRIDES[i]
        wbig = _fold_w_packed(w, g)
        bt = jnp.tile(b, 8).reshape(1, 8 * C)
        xpad = jnp.pad(h, ((1, 1), (0, 0), (0, 0)))
        D_out = D if s == 1 else D // 2
        y = _conv_packed(xpad, wbig, bt, D_out, H, W // 8, s)
        if s == 2:
            y = y.reshape(D_out, H, W // 8, 8, C)[:, ::2, :, ::2]
            H //= 2
            W //= 2
            y = y.reshape(D_out, H * (W // 8), 8 * C)
        D = D_out
        h = y
        if i == 4:
            t = h.reshape(D, H, W // 8, 8, C).reshape(D, H, W, C)
            outs.append(jnp.transpose(t, (3, 0, 1, 2))[None])
    w_all = jnp.stack([_fold_w(params[i][0], params[i][1]) for i in range(6, 17)])
    b_all = jnp.stack([params[i][2].reshape(1, C) for i in range(6, 17)])
    x16 = h.reshape(16, 16, 2, 8, C).reshape(16 ** 3, C)
    o2, o3, o4 = _fused_tail(x16, w_all, b_all)
    outs.append(jnp.transpose(o2.reshape(16, 16, 16, C), (3, 0, 1, 2))[None])
    outs.append(jnp.transpose(o3.reshape(8, 8, 8, C), (3, 0, 1, 2))[None])
    outs.append(jnp.transpose(o4.reshape(4, 4, 4, C), (3, 0, 1, 2))[None])
    return outs[0], outs[1], outs[2], outs[3]
